# baseline (speedup 1.0000x reference)
import functools

import jax
import jax.numpy as jnp
from jax import lax
from jax.experimental import pallas as pl
from jax.experimental.pallas import tpu as pltpu
from jax.experimental.pallas import tpu_sc as plsc

N_NODES = 10000
N_EDGES = 320000
D_FEAT = 128
PHASE_DIM = 128

_INFO = plsc.get_sparse_core_info()
_NC = _INFO.num_cores
_NS = _INFO.num_subcores
_NW = _NC * _NS
_EPW = N_EDGES // _NW
_C = 128
_H = _C // 2
_NFULL = _EPW // _C
_TAIL = _EPW - _NFULL * _C
_RROWS = 2000


def _sc_body(table, edges, out2, idx1_v, idx0_v,
             r2a, r2b, r2c, gsa, gsb, gsc, wsa, wsb, wsc):
    w = lax.axis_index("s") * _NC + lax.axis_index("c")
    base = w * _EPW
    pltpu.sync_copy(edges.at[pl.ds(N_EDGES + base, _EPW)], idx1_v)
    pltpu.sync_copy(edges.at[pl.ds(base, _EPW)], idx0_v)

    def issue_g(j, r2, gs):
        off = j * _C
        for h in (0, _H):
            pltpu.async_copy(table.at[idx1_v.at[pl.ds(off + h, _H)]],
                             r2.at[pl.ds(h, _H), pl.ds(0, D_FEAT)], gs)
            pltpu.async_copy(table.at[idx0_v.at[pl.ds(off + h, _H)]],
                             r2.at[pl.ds(h, _H), pl.ds(D_FEAT, D_FEAT)], gs)

    def wait_g(r2, gs):
        for h in (0, _H):
            pltpu.make_async_copy(table.at[idx1_v.at[pl.ds(h, _H)]],
                                  r2.at[pl.ds(h, _H), pl.ds(0, D_FEAT)],
                                  gs).wait()
            pltpu.make_async_copy(table.at[idx0_v.at[pl.ds(h, _H)]],
                                  r2.at[pl.ds(h, _H), pl.ds(D_FEAT, D_FEAT)],
                                  gs).wait()

    def issue_w(j, r2, ws):
        pltpu.async_copy(r2, out2.at[pl.ds(base + j * _C, _C)], ws)

    def wait_w(r2, ws):
        pltpu.make_async_copy(r2, out2.at[pl.ds(base, _C)], ws).wait()

    issue_g(0, r2a, gsa)
    issue_g(1, r2b, gsb)
    issue_g(2, r2c, gsc)

    def outer(jj3, carry):
        jj = jj3 * 3
        wait_g(r2a, gsa)
        issue_w(jj, r2a, wsa)
        wait_g(r2b, gsb)
        issue_w(jj + 1, r2b, wsb)
        wait_g(r2c, gsc)
        issue_w(jj + 2, r2c, wsc)
        wait_w(r2a, wsa)
        issue_g(jj + 3, r2a, gsa)
        wait_w(r2b, wsb)
        issue_g(jj + 4, r2b, gsb)
        wait_w(r2c, wsc)
        issue_g(jj + 5, r2c, gsc)
        return carry

    lax.fori_loop(0, _NFULL // 3 - 1, outer, 0)

    jj = _NFULL - 3
    wait_g(r2a, gsa)
    issue_w(jj, r2a, wsa)
    wait_g(r2b, gsb)
    issue_w(jj + 1, r2b, wsb)
    wait_g(r2c, gsc)
    issue_w(jj + 2, r2c, wsc)
    wait_w(r2a, wsa)
    wait_w(r2b, wsb)
    wait_w(r2c, wsc)

    toff = _NFULL * _C
    tg = base + toff
    c2 = pltpu.async_copy(table.at[idx1_v.at[pl.ds(toff, _TAIL)]],
                          r2a.at[pl.ds(0, _TAIL), pl.ds(0, D_FEAT)], gsa)
    c3 = pltpu.async_copy(table.at[idx0_v.at[pl.ds(toff, _TAIL)]],
                          r2a.at[pl.ds(0, _TAIL), pl.ds(D_FEAT, D_FEAT)], gsa)
    c2.wait()
    c3.wait()
    pltpu.sync_copy(r2a.at[pl.ds(0, _TAIL)], out2.at[pl.ds(tg, _TAIL)])


def _tc_body(comp_ref, rel_ref, out_ref):
    cf = comp_ref[...].astype(jnp.float32)
    r0 = rel_ref[0:1, :]
    r1 = rel_ref[1:2, :]
    out_ref[...] = cf * r1 + (1.0 - cf) * r0


@jax.jit
def _run(table, comp, edges, rel):
    out1 = pl.pallas_call(
        _tc_body,
        grid=(N_EDGES // _RROWS,),
        in_specs=[
            pl.BlockSpec((_RROWS, 1), lambda g: (g, 0)),
            pl.BlockSpec((2, PHASE_DIM), lambda g: (0, 0)),
        ],
        out_specs=pl.BlockSpec((_RROWS, PHASE_DIM), lambda g: (g, 0)),
        out_shape=jax.ShapeDtypeStruct((N_EDGES, PHASE_DIM), jnp.float32),
    )(comp.reshape(N_EDGES, 1), rel)

    mesh = plsc.VectorSubcoreMesh(core_axis_name="c", subcore_axis_name="s")
    sc = functools.partial(
        pl.kernel,
        mesh=mesh,
        out_type=jax.ShapeDtypeStruct((N_EDGES, 2 * D_FEAT), jnp.float32),
        scratch_types=[
            pltpu.VMEM((_EPW,), jnp.int32),
            pltpu.VMEM((_EPW,), jnp.int32),
            pltpu.VMEM((_C, 2 * D_FEAT), jnp.float32),
            pltpu.VMEM((_C, 2 * D_FEAT), jnp.float32),
            pltpu.VMEM((_C, 2 * D_FEAT), jnp.float32),
            pltpu.SemaphoreType.DMA,
            pltpu.SemaphoreType.DMA,
            pltpu.SemaphoreType.DMA,
            pltpu.SemaphoreType.DMA,
            pltpu.SemaphoreType.DMA,
            pltpu.SemaphoreType.DMA,
        ],
    )(_sc_body)
    out2 = sc(table, edges)
    return out1, out2


def kernel(phase_demand_embedding, pair_partial_competing, pair_edge_index,
           pair_relation_table):
    out1, out2 = _run(phase_demand_embedding, pair_partial_competing,
                      pair_edge_index.reshape(-1), pair_relation_table)
    return (out1, out2)


# speedup vs baseline: 1.3876x; 1.3876x over previous
import functools

import jax
import jax.numpy as jnp
from jax import lax
from jax.experimental import pallas as pl
from jax.experimental.pallas import tpu as pltpu
from jax.experimental.pallas import tpu_sc as plsc

N_NODES = 10000
N_EDGES = 320000
D_FEAT = 128
PHASE_DIM = 128

_INFO = plsc.get_sparse_core_info()
_NC = _INFO.num_cores
_NS = _INFO.num_subcores
_L = _INFO.num_lanes
_NW = _NC * _NS
_EPW = N_EDGES // _NW
_C = 128
_NFULL = _EPW // _C
_TAIL = _EPW - _NFULL * _C
_NB = PHASE_DIM // _L


def _body(table, comp, edges, rel, out1, out2,
          idx1_v, idx0_v, comp_v, rel_v, r1a, r2a, r1b, r2b,
          gsa, gsb, wsa, wsb):
    w = lax.axis_index("s") * _NC + lax.axis_index("c")
    base = w * _EPW
    pltpu.sync_copy(edges.at[pl.ds(N_EDGES + base, _EPW)], idx1_v)
    pltpu.sync_copy(edges.at[pl.ds(base, _EPW)], idx0_v)
    pltpu.sync_copy(comp.at[pl.ds(base, _EPW)], comp_v)
    pltpu.sync_copy(rel, rel_v)

    r0s = [rel_v[pl.ds(b * _L, _L)] for b in range(_NB)]
    d_s = [rel_v[pl.ds(PHASE_DIM + b * _L, _L)] - r0s[b] for b in range(_NB)]

    def compute_rel(j, r1, n):
        off = j * _C

        def group(k, carry):
            cf = comp_v[pl.ds(off + k * _L, _L)].astype(jnp.float32)

            def row(i, carry2):
                c = lax.gather(
                    cf, jnp.full((_L, 1), i, jnp.int32),
                    dimension_numbers=lax.GatherDimensionNumbers(
                        offset_dims=(), collapsed_slice_dims=(0,),
                        start_index_map=(0,)),
                    slice_sizes=(1,),
                    mode=lax.GatherScatterMode.PROMISE_IN_BOUNDS)
                rbase = (k * _L + i) * PHASE_DIM
                for b in range(_NB):
                    r1[pl.ds(rbase + b * _L, _L)] = r0s[b] + c * d_s[b]
                return carry2

            return lax.fori_loop(0, _L, row, carry)

        lax.fori_loop(0, n // _L, group, 0)

    def issue_g(j, r2, gs):
        off = j * _C
        pltpu.async_copy(table.at[idx1_v.at[pl.ds(off, _C)]],
                         r2.at[:, pl.ds(0, D_FEAT)], gs)
        pltpu.async_copy(table.at[idx0_v.at[pl.ds(off, _C)]],
                         r2.at[:, pl.ds(D_FEAT, D_FEAT)], gs)

    def wait_g(r2, gs):
        pltpu.make_async_copy(table.at[idx1_v.at[pl.ds(0, _C)]],
                              r2.at[:, pl.ds(0, D_FEAT)], gs).wait()
        pltpu.make_async_copy(table.at[idx0_v.at[pl.ds(0, _C)]],
                              r2.at[:, pl.ds(D_FEAT, D_FEAT)], gs).wait()

    def issue_w(j, r1, r2, ws):
        g = base + j * _C
        pltpu.async_copy(r1, out1.at[pl.ds(g * PHASE_DIM, _C * PHASE_DIM)], ws)
        pltpu.async_copy(r2, out2.at[pl.ds(g, _C)], ws)

    def wait_w(r1, r2, ws):
        pltpu.make_async_copy(r1, out1.at[pl.ds(0, _C * PHASE_DIM)], ws).wait()
        pltpu.make_async_copy(r2, out2.at[pl.ds(base, _C)], ws).wait()

    issue_g(0, r2a, gsa)
    issue_g(1, r2b, gsb)
    compute_rel(0, r1a, _C)

    def outer(jj2, carry):
        jj = jj2 * 2
        wait_g(r2a, gsa)
        issue_w(jj, r1a, r2a, wsa)
        compute_rel(jj + 1, r1b, _C)
        wait_g(r2b, gsb)
        issue_w(jj + 1, r1b, r2b, wsb)
        wait_w(r1a, r2a, wsa)
        issue_g(jj + 2, r2a, gsa)
        compute_rel(jj + 2, r1a, _C)
        wait_w(r1b, r2b, wsb)
        issue_g(jj + 3, r2b, gsb)
        return carry

    lax.fori_loop(0, _NFULL // 2 - 1, outer, 0)

    jj = _NFULL - 2
    wait_g(r2a, gsa)
    issue_w(jj, r1a, r2a, wsa)
    compute_rel(jj + 1, r1b, _C)
    wait_g(r2b, gsb)
    issue_w(jj + 1, r1b, r2b, wsb)
    wait_w(r1a, r2a, wsa)
    wait_w(r1b, r2b, wsb)

    toff = _NFULL * _C
    tg = base + toff
    c2 = pltpu.async_copy(table.at[idx1_v.at[pl.ds(toff, _TAIL)]],
                          r2a.at[pl.ds(0, _TAIL), pl.ds(0, D_FEAT)], gsa)
    c3 = pltpu.async_copy(table.at[idx0_v.at[pl.ds(toff, _TAIL)]],
                          r2a.at[pl.ds(0, _TAIL), pl.ds(D_FEAT, D_FEAT)], gsa)
    compute_rel(_NFULL, r1a, _TAIL)
    c2.wait()
    c3.wait()
    pltpu.sync_copy(r1a.at[pl.ds(0, _TAIL * PHASE_DIM)],
                    out1.at[pl.ds(tg * PHASE_DIM, _TAIL * PHASE_DIM)])
    pltpu.sync_copy(r2a.at[pl.ds(0, _TAIL)], out2.at[pl.ds(tg, _TAIL)])


@jax.jit
def _run(table, comp, edges, rel):
    mesh = plsc.VectorSubcoreMesh(core_axis_name="c", subcore_axis_name="s")
    f = functools.partial(
        pl.kernel,
        mesh=mesh,
        out_type=[
            jax.ShapeDtypeStruct((N_EDGES * PHASE_DIM,), jnp.float32),
            jax.ShapeDtypeStruct((N_EDGES, 2 * D_FEAT), jnp.float32),
        ],
        scratch_types=[
            pltpu.VMEM((_EPW,), jnp.int32),
            pltpu.VMEM((_EPW,), jnp.int32),
            pltpu.VMEM((_EPW,), jnp.int32),
            pltpu.VMEM((2 * PHASE_DIM,), jnp.float32),
            pltpu.VMEM((_C * PHASE_DIM,), jnp.float32),
            pltpu.VMEM((_C, 2 * D_FEAT), jnp.float32),
            pltpu.VMEM((_C * PHASE_DIM,), jnp.float32),
            pltpu.VMEM((_C, 2 * D_FEAT), jnp.float32),
            pltpu.SemaphoreType.DMA,
            pltpu.SemaphoreType.DMA,
            pltpu.SemaphoreType.DMA,
            pltpu.SemaphoreType.DMA,
        ],
    )(_body)
    return f(table, comp, edges, rel)


def kernel(phase_demand_embedding, pair_partial_competing, pair_edge_index,
           pair_relation_table):
    out1, out2 = _run(phase_demand_embedding, pair_partial_competing,
                      pair_edge_index.reshape(-1),
                      pair_relation_table.reshape(-1))
    return (out1.reshape(N_EDGES, PHASE_DIM), out2)
